# baseline (speedup 1.0000x reference)
import functools

import jax
import jax.numpy as jnp
from jax.experimental import pallas as pl


def _reduce_body(x_ref, out_ref):
    x = x_ref[...] - 0.5
    ps = jnp.sum(x, axis=(0, 1))
    ps2 = jnp.sum(x * x, axis=(0, 1))
    out_ref[...] = jnp.stack([ps, ps2]).reshape(1, 2, -1)


def _normalize_body(n_total, n_opes, n_mas,
                    proc_ref, part_ref, opes_ref, mas_ref,
                    proc_out, opes_out, mas_out):
    parts = part_ref[...]
    s = jnp.sum(parts[:, 0, :])
    s2 = jnp.sum(parts[:, 1, :])
    n = float(n_total)
    gvar = (s2 - s * s / n) / (n - 1.0)
    ginv = 1.0 / (jnp.sqrt(gvar) + 1e-5)
    gmean = 0.5 + s / n
    proc_out[...] = (proc_ref[...] - gmean) * ginv

    x = opes_ref[...]
    m = jnp.mean(x, axis=1, keepdims=True)
    d = x - m
    v = jnp.sum(d * d, axis=1, keepdims=True) * (1.0 / (n_opes - 1.0))
    opes_out[...] = d / (jnp.sqrt(v) + 1e-5)

    y = mas_ref[...]
    my = jnp.mean(y, axis=1, keepdims=True)
    dy = y - my
    vy = jnp.sum(dy * dy, axis=1, keepdims=True) * (1.0 / (n_mas - 1.0))
    mas_out[...] = dy / (jnp.sqrt(vy) + 1e-5)


def kernel(raw_opes, raw_mas, proc_time, batch_idxes, nums_opes):
    B, N_OPES, D_OPE = raw_opes.shape
    _, N_MAS, D_MA = raw_mas.shape
    BS = 2
    G = B // BS
    n_total = B * N_OPES * N_MAS

    partials = pl.pallas_call(
        _reduce_body,
        grid=(G,),
        in_specs=[pl.BlockSpec((BS, N_OPES, N_MAS), lambda i: (i, 0, 0))],
        out_specs=pl.BlockSpec((1, 2, N_MAS), lambda i: (i, 0, 0)),
        out_shape=jax.ShapeDtypeStruct((G, 2, N_MAS), jnp.float32),
    )(proc_time)

    proc_norm, opes_norm, mas_norm = pl.pallas_call(
        functools.partial(_normalize_body, n_total, N_OPES, N_MAS),
        grid=(G,),
        in_specs=[
            pl.BlockSpec((BS, N_OPES, N_MAS), lambda i: (i, 0, 0)),
            pl.BlockSpec((G, 2, N_MAS), lambda i: (0, 0, 0)),
            pl.BlockSpec((BS, N_OPES, D_OPE), lambda i: (i, 0, 0)),
            pl.BlockSpec((BS, N_MAS, D_MA), lambda i: (i, 0, 0)),
        ],
        out_specs=[
            pl.BlockSpec((BS, N_OPES, N_MAS), lambda i: (i, 0, 0)),
            pl.BlockSpec((BS, N_OPES, D_OPE), lambda i: (i, 0, 0)),
            pl.BlockSpec((BS, N_MAS, D_MA), lambda i: (i, 0, 0)),
        ],
        out_shape=[
            jax.ShapeDtypeStruct((B, N_OPES, N_MAS), jnp.float32),
            jax.ShapeDtypeStruct((B, N_OPES, D_OPE), jnp.float32),
            jax.ShapeDtypeStruct((B, N_MAS, D_MA), jnp.float32),
        ],
    )(proc_time, partials, raw_opes, raw_mas)

    return (opes_norm, mas_norm, proc_norm)


# speedup vs baseline: 1.3917x; 1.3917x over previous
import functools

import jax
import jax.numpy as jnp
from jax.experimental import pallas as pl

_LANES = 4096
_BR = 256


def _reduce_body(x_ref, out_ref):
    x = x_ref[...] - 0.5
    ps = jnp.sum(x, axis=0)
    ps2 = jnp.sum(x * x, axis=0)
    out_ref[...] = jnp.stack([ps, ps2]).reshape(1, 2, -1)


def _masked_normalize(x, n_feat, n_red):
    fid = jax.lax.broadcasted_iota(jnp.int32, x.shape, dimension=1) % n_feat
    inv_n = 1.0 / float(n_red)
    mean_map = jnp.zeros_like(x)
    masks = []
    for f in range(n_feat):
        mf = (fid == f).astype(x.dtype)
        masks.append(mf)
        mean_map += (jnp.sum(x * mf, axis=1, keepdims=True) * inv_n) * mf
    d = x - mean_map
    inv_dd = 1.0 / float(n_red - 1)
    inv_map = jnp.zeros_like(x)
    for f in range(n_feat):
        mf = masks[f]
        vf = jnp.sum(d * d * mf, axis=1, keepdims=True) * inv_dd
        inv_map += mf / (jnp.sqrt(vf) + 1e-5)
    return d * inv_map


def _normalize_body(n_total, n_opes, n_mas,
                    proc_ref, part_ref, opes_ref, mas_ref,
                    proc_out, opes_out, mas_out):
    parts = part_ref[...]
    s = jnp.sum(parts[:, 0, :])
    s2 = jnp.sum(parts[:, 1, :])
    n = float(n_total)
    gvar = (s2 - s * s / n) / (n - 1.0)
    ginv = 1.0 / (jnp.sqrt(gvar) + 1e-5)
    gmean = 0.5 + s / n
    proc_out[...] = (proc_ref[...] - gmean) * ginv

    opes_out[...] = _masked_normalize(opes_ref[...], 4, n_opes)
    mas_out[...] = _masked_normalize(mas_ref[...], 2, n_mas)


def kernel(raw_opes, raw_mas, proc_time, batch_idxes, nums_opes):
    B, N_OPES, D_OPE = raw_opes.shape
    _, N_MAS, D_MA = raw_mas.shape
    n_total = B * N_OPES * N_MAS
    rows = n_total // _LANES
    G = rows // _BR
    bs = B // G

    proc2d = proc_time.reshape(rows, _LANES)
    opes2d = raw_opes.reshape(B, N_OPES * D_OPE)
    mas2d = raw_mas.reshape(B, N_MAS * D_MA)

    partials = pl.pallas_call(
        _reduce_body,
        grid=(G,),
        in_specs=[pl.BlockSpec((_BR, _LANES), lambda i: (i, 0))],
        out_specs=pl.BlockSpec((1, 2, _LANES), lambda i: (i, 0, 0)),
        out_shape=jax.ShapeDtypeStruct((G, 2, _LANES), jnp.float32),
    )(proc2d)

    proc_norm, opes_norm, mas_norm = pl.pallas_call(
        functools.partial(_normalize_body, n_total, N_OPES, N_MAS),
        grid=(G,),
        in_specs=[
            pl.BlockSpec((_BR, _LANES), lambda i: (i, 0)),
            pl.BlockSpec((G, 2, _LANES), lambda i: (0, 0, 0)),
            pl.BlockSpec((bs, N_OPES * D_OPE), lambda i: (i, 0)),
            pl.BlockSpec((bs, N_MAS * D_MA), lambda i: (i, 0)),
        ],
        out_specs=[
            pl.BlockSpec((_BR, _LANES), lambda i: (i, 0)),
            pl.BlockSpec((bs, N_OPES * D_OPE), lambda i: (i, 0)),
            pl.BlockSpec((bs, N_MAS * D_MA), lambda i: (i, 0)),
        ],
        out_shape=[
            jax.ShapeDtypeStruct((rows, _LANES), jnp.float32),
            jax.ShapeDtypeStruct((B, N_OPES * D_OPE), jnp.float32),
            jax.ShapeDtypeStruct((B, N_MAS * D_MA), jnp.float32),
        ],
    )(proc2d, partials, opes2d, mas2d)

    return (opes_norm.reshape(B, N_OPES, D_OPE),
            mas_norm.reshape(B, N_MAS, D_MA),
            proc_norm.reshape(B, N_OPES, N_MAS))


# speedup vs baseline: 7.8487x; 5.6395x over previous
import functools

import jax
import jax.numpy as jnp
from jax.experimental import pallas as pl
from jax.experimental.pallas import tpu as pltpu

_BS = 8


def _body(g, n_total, n_opes, n_mas,
          proc_ref, opes_ref, mas_ref,
          proc_out, opes_out, mas_out, acc_ref):
    i = pl.program_id(0)

    @pl.when(i == 0)
    def _init():
        acc_ref[...] = jnp.zeros_like(acc_ref)
        y = mas_ref[...]
        my = jnp.mean(y, axis=0, keepdims=True)
        dy = y - my
        vy = jnp.sum(dy * dy, axis=0, keepdims=True) * (1.0 / (n_mas - 1.0))
        mas_out[...] = dy / (jnp.sqrt(vy) + 1e-5)

    @pl.when(i < g)
    def _phase0():
        x = proc_ref[...] - 0.5
        ps = jnp.sum(x, axis=(0, 1))
        ps2 = jnp.sum(x * x, axis=(0, 1))
        acc_ref[...] += jnp.stack([ps, ps2])

        z = opes_ref[...]
        m = jnp.mean(z, axis=2, keepdims=True)
        d = z - m
        v = jnp.sum(d * d, axis=2, keepdims=True) * (1.0 / (n_opes - 1.0))
        opes_out[...] = d / (jnp.sqrt(v) + 1e-5)

    @pl.when(i >= g)
    def _phase1():
        acc = acc_ref[...]
        s = jnp.sum(acc[0:1, :])
        s2 = jnp.sum(acc[1:2, :])
        n = float(n_total)
        gvar = (s2 - s * s / n) / (n - 1.0)
        ginv = 1.0 / (jnp.sqrt(gvar) + 1e-5)
        gmean = s / n
        proc_out[...] = ((proc_ref[...] - 0.5) - gmean) * ginv


def kernel(raw_opes, raw_mas, proc_time, batch_idxes, nums_opes):
    B, N_OPES, D_OPE = raw_opes.shape
    _, N_MAS, D_MA = raw_mas.shape
    n_total = B * N_OPES * N_MAS
    G = B // _BS
    bs = B // G

    pt = jnp.transpose(proc_time, (0, 2, 1))
    ot = jnp.transpose(raw_opes, (0, 2, 1))
    mt = jnp.transpose(raw_mas, (1, 2, 0))

    pn, on, mn = pl.pallas_call(
        functools.partial(_body, G, n_total, N_OPES, N_MAS),
        grid=(2 * G,),
        in_specs=[
            pl.BlockSpec((_BS, N_MAS, N_OPES), lambda i: (i % G, 0, 0)),
            pl.BlockSpec((bs, D_OPE, N_OPES),
                         lambda i: (jnp.minimum(i, G - 1), 0, 0)),
            pl.BlockSpec((N_MAS, D_MA, B), lambda i: (0, 0, 0)),
        ],
        out_specs=[
            pl.BlockSpec((_BS, N_MAS, N_OPES),
                         lambda i: (jnp.maximum(i - G, 0), 0, 0)),
            pl.BlockSpec((bs, D_OPE, N_OPES),
                         lambda i: (jnp.minimum(i, G - 1), 0, 0)),
            pl.BlockSpec((N_MAS, D_MA, B), lambda i: (0, 0, 0)),
        ],
        out_shape=[
            jax.ShapeDtypeStruct((B, N_MAS, N_OPES), jnp.float32),
            jax.ShapeDtypeStruct((B, D_OPE, N_OPES), jnp.float32),
            jax.ShapeDtypeStruct((N_MAS, D_MA, B), jnp.float32),
        ],
        scratch_shapes=[pltpu.VMEM((2, N_OPES), jnp.float32)],
    )(pt, ot, mt)

    return (jnp.transpose(on, (0, 2, 1)),
            jnp.transpose(mn, (2, 0, 1)),
            jnp.transpose(pn, (0, 2, 1)))


# speedup vs baseline: 8.4713x; 1.0793x over previous
import functools

import jax
import jax.numpy as jnp
from jax.experimental import pallas as pl
from jax.experimental.pallas import tpu as pltpu

_BS = 16


def _body(g, n_total, n_opes, n_mas,
          proc_ref, opes_ref, mas_ref,
          proc_out, opes_out, mas_out, acc_ref):
    i = pl.program_id(0)

    @pl.when(i == 0)
    def _init():
        acc_ref[...] = jnp.zeros_like(acc_ref)
        y = mas_ref[...]
        my = jnp.mean(y, axis=0, keepdims=True)
        dy = y - my
        vy = jnp.sum(dy * dy, axis=0, keepdims=True) * (1.0 / (n_mas - 1.0))
        mas_out[...] = dy / (jnp.sqrt(vy) + 1e-5)

    @pl.when(i < g)
    def _phase0():
        x = proc_ref[...] - 0.5
        ps = jnp.sum(x, axis=(0, 1))
        ps2 = jnp.sum(x * x, axis=(0, 1))
        acc_ref[...] += jnp.stack([ps, ps2])

        z = opes_ref[...]
        m = jnp.mean(z, axis=2, keepdims=True)
        d = z - m
        v = jnp.sum(d * d, axis=2, keepdims=True) * (1.0 / (n_opes - 1.0))
        opes_out[...] = d / (jnp.sqrt(v) + 1e-5)

    @pl.when(i >= g)
    def _phase1():
        acc = acc_ref[...]
        s = jnp.sum(acc[0:1, :])
        s2 = jnp.sum(acc[1:2, :])
        n = float(n_total)
        gvar = (s2 - s * s / n) / (n - 1.0)
        ginv = 1.0 / (jnp.sqrt(gvar) + 1e-5)
        gmean = s / n
        proc_out[...] = ((proc_ref[...] - 0.5) - gmean) * ginv


def kernel(raw_opes, raw_mas, proc_time, batch_idxes, nums_opes):
    B, N_OPES, D_OPE = raw_opes.shape
    _, N_MAS, D_MA = raw_mas.shape
    n_total = B * N_OPES * N_MAS
    G = B // _BS
    bs = B // G

    pt = jnp.transpose(proc_time, (0, 2, 1))
    ot = jnp.transpose(raw_opes, (0, 2, 1))
    mt = jnp.transpose(raw_mas, (1, 2, 0))

    pn, on, mn = pl.pallas_call(
        functools.partial(_body, G, n_total, N_OPES, N_MAS),
        grid=(2 * G,),
        in_specs=[
            pl.BlockSpec((_BS, N_MAS, N_OPES), lambda i: (i % G, 0, 0)),
            pl.BlockSpec((bs, D_OPE, N_OPES),
                         lambda i: (jnp.minimum(i, G - 1), 0, 0)),
            pl.BlockSpec((N_MAS, D_MA, B), lambda i: (0, 0, 0)),
        ],
        out_specs=[
            pl.BlockSpec((_BS, N_MAS, N_OPES),
                         lambda i: (jnp.maximum(i - G, 0), 0, 0)),
            pl.BlockSpec((bs, D_OPE, N_OPES),
                         lambda i: (jnp.minimum(i, G - 1), 0, 0)),
            pl.BlockSpec((N_MAS, D_MA, B), lambda i: (0, 0, 0)),
        ],
        out_shape=[
            jax.ShapeDtypeStruct((B, N_MAS, N_OPES), jnp.float32),
            jax.ShapeDtypeStruct((B, D_OPE, N_OPES), jnp.float32),
            jax.ShapeDtypeStruct((N_MAS, D_MA, B), jnp.float32),
        ],
        scratch_shapes=[pltpu.VMEM((2, N_OPES), jnp.float32)],
    )(pt, ot, mt)

    return (jnp.transpose(on, (0, 2, 1)),
            jnp.transpose(mn, (2, 0, 1)),
            jnp.transpose(pn, (0, 2, 1)))
